# Initial kernel scaffold; baseline (speedup 1.0000x reference)
import jax
import jax.numpy as jnp
from jax.experimental import pallas as pl


def kernel(x, edge_index, node_graph_index, W1_0, b1_0, W2_0, b2_0, gam_0, bet_0, mu_0, var_0, W1_1, b1_1, W2_1, b2_1, gam_1, bet_1, mu_1, var_1, W1_2, b1_2, W2_2, b2_2, gam_2, bet_2, mu_2, var_2, Wm1, bm1, Wm2, bm2):
    raise NotImplementedError("write your pallas kernel here")


# speedup vs baseline: 4.5344x; 4.5344x over previous
import functools

import jax
import jax.numpy as jnp
from jax import lax
from jax.experimental import pallas as pl
from jax.experimental.pallas import tpu as pltpu
from jax.experimental.pallas import tpu_sc as plsc

N = 10000
E = 320000
D = 128
G = 64
C = 2

NC = 2
NS = 16
NW = NC * NS
EPW = E // NW
K = 80
CHUNKS = EPW // K
RPT = 624
TAIL = N - NS * RPT
TAIL_OFF = NS * RPT


def _sc_scatter_body(h_hbm, src_hbm, dst_hbm, zero_hbm, out_hbm,
                     src_v, dst_v, rows_v, agg_sh, sem):
    c = lax.axis_index("c")
    s = lax.axis_index("s")
    pltpu.sync_copy(zero_hbm.at[pl.ds(s * RPT, RPT)],
                    agg_sh.at[pl.ds(s * RPT, RPT)])

    @pl.when(s == NS - 1)
    def _():
        pltpu.sync_copy(zero_hbm.at[pl.ds(TAIL_OFF, TAIL)],
                        agg_sh.at[pl.ds(TAIL_OFF, TAIL)])

    plsc.subcore_barrier()
    base = (s * NC + c) * EPW

    def body(i, carry):
        off = base + i * K
        pltpu.sync_copy(src_hbm.at[pl.ds(off, K)], src_v)
        pltpu.sync_copy(dst_hbm.at[pl.ds(off, K)], dst_v)
        pltpu.async_copy(h_hbm.at[src_v], rows_v, sem).wait()
        pltpu.sync_copy(rows_v, agg_sh.at[dst_v], add=True)
        return carry

    lax.fori_loop(0, CHUNKS, body, 0)
    plsc.subcore_barrier()
    pltpu.sync_copy(agg_sh.at[pl.ds(s * RPT, RPT)],
                    out_hbm.at[c, pl.ds(s * RPT, RPT)])

    @pl.when(s == NS - 1)
    def _():
        pltpu.sync_copy(agg_sh.at[pl.ds(TAIL_OFF, TAIL)],
                        out_hbm.at[c, pl.ds(TAIL_OFF, TAIL)])


@functools.cache
def _sc_scatter():
    return functools.partial(
        pl.kernel,
        mesh=plsc.VectorSubcoreMesh(core_axis_name="c", subcore_axis_name="s"),
        out_type=jax.ShapeDtypeStruct((NC, N, D), jnp.float32),
        scratch_types=[
            pltpu.VMEM((K,), jnp.int32),
            pltpu.VMEM((K,), jnp.int32),
            pltpu.VMEM((K, D), jnp.float32),
            pltpu.VMEM_SHARED((N, D), jnp.float32),
            pltpu.SemaphoreType.DMA,
        ],
    )(_sc_scatter_body)


BLK = 1000
NBLK = N // BLK


def _tc_layer_body(ngi_ref, h_ref, a0_ref, a1_ref, w1_ref, b1_ref,
                   w2_ref, b2_ref, hout_ref, pool_ref):
    i = pl.program_id(0)
    hin = h_ref[...] + a0_ref[...] + a1_ref[...]
    t = jnp.dot(hin, w1_ref[...], preferred_element_type=jnp.float32)
    t = jnp.maximum(t + b1_ref[...], 0.0)
    t = jnp.dot(t, w2_ref[...], preferred_element_type=jnp.float32)
    t = jnp.maximum(t + b2_ref[...], 0.0)
    hout_ref[...] = t
    ngi = ngi_ref[0, 0, :]
    onehot_t = (lax.broadcasted_iota(jnp.int32, (G, BLK), 0)
                == ngi[None, :]).astype(jnp.float32)
    part = jnp.dot(onehot_t, t, preferred_element_type=jnp.float32)

    @pl.when(i == 0)
    def _():
        pool_ref[...] = part

    @pl.when(i != 0)
    def _():
        pool_ref[...] += part


_tc_layer = pl.pallas_call(
    _tc_layer_body,
    grid=(NBLK,),
    in_specs=[
        pl.BlockSpec((1, 1, BLK), lambda i: (i, 0, 0)),
        pl.BlockSpec((BLK, D), lambda i: (i, 0)),
        pl.BlockSpec((BLK, D), lambda i: (i, 0)),
        pl.BlockSpec((BLK, D), lambda i: (i, 0)),
        pl.BlockSpec((D, D), lambda i: (0, 0)),
        pl.BlockSpec((1, D), lambda i: (0, 0)),
        pl.BlockSpec((D, D), lambda i: (0, 0)),
        pl.BlockSpec((1, D), lambda i: (0, 0)),
    ],
    out_specs=[
        pl.BlockSpec((BLK, D), lambda i: (i, 0)),
        pl.BlockSpec((G, D), lambda i: (0, 0)),
    ],
    out_shape=[
        jax.ShapeDtypeStruct((N, D), jnp.float32),
        jax.ShapeDtypeStruct((G, D), jnp.float32),
    ],
)


def _tc_head_body(p0_ref, p1_ref, p2_ref, wm1_ref, bm1_ref, wm2_ref,
                  bm2_ref, out_ref):
    hm = (jnp.dot(p0_ref[...], wm1_ref[0:D, :],
                  preferred_element_type=jnp.float32)
          + jnp.dot(p1_ref[...], wm1_ref[D:2 * D, :],
                    preferred_element_type=jnp.float32)
          + jnp.dot(p2_ref[...], wm1_ref[2 * D:3 * D, :],
                    preferred_element_type=jnp.float32))
    hm = jnp.maximum(hm + bm1_ref[...], 0.0)
    out_ref[...] = (jnp.dot(hm, wm2_ref[...],
                            preferred_element_type=jnp.float32)
                    + bm2_ref[...])


_tc_head = pl.pallas_call(
    _tc_head_body,
    out_shape=jax.ShapeDtypeStruct((G, 128), jnp.float32),
)


def kernel(x, edge_index, node_graph_index,
           W1_0, b1_0, W2_0, b2_0, gam_0, bet_0, mu_0, var_0,
           W1_1, b1_1, W2_1, b2_1, gam_1, bet_1, mu_1, var_1,
           W1_2, b1_2, W2_2, b2_2, gam_2, bet_2, mu_2, var_2,
           Wm1, bm1, Wm2, bm2):
    src = edge_index[0]
    dst = edge_index[1]
    zero = jnp.zeros((N, D), jnp.float32)
    ngi_r = node_graph_index.reshape(NBLK, 1, BLK)

    layer_params = [
        (W1_0, b1_0, W2_0, b2_0, gam_0, bet_0, mu_0, var_0),
        (W1_1, b1_1, W2_1, b2_1, gam_1, bet_1, mu_1, var_1),
        (W1_2, b1_2, W2_2, b2_2, gam_2, bet_2, mu_2, var_2),
    ]

    h = x
    pools = []
    for (W1, b1, W2, b2, gam, bet, mu, var) in layer_params:
        s = gam / jnp.sqrt(var + 1e-3)
        W2f = W2 * s[None, :]
        b2f = b2 * s + bet - mu * s
        agg = _sc_scatter()(h, src, dst, zero)
        h, pool = _tc_layer(ngi_r, h, agg[0], agg[1],
                            W1, b1.reshape(1, D), W2f, b2f.reshape(1, D))
        pools.append(pool)

    Wm2p = jnp.zeros((128, 128), jnp.float32).at[:, :C].set(Wm2)
    bm2p = jnp.zeros((1, 128), jnp.float32).at[0, :C].set(bm2)
    out = _tc_head(pools[0], pools[1], pools[2],
                   Wm1, bm1.reshape(1, 128), Wm2p, bm2p)
    return out[:, :C]
